# initial kernel scaffold (unmeasured)
import jax
import jax.numpy as jnp
from jax import lax
from jax.experimental import pallas as pl
from jax.experimental.pallas import tpu as pltpu

N_DEV = 32
B, SQ, D_MODEL = 2, 256, 768
H_LOC = 8
DH = 64
N_ROWS = B * SQ
CHUNK = N_ROWS // N_DEV


def kernel(x, Wq, Wo, K_ext, V_ext):
    my = lax.axis_index("i")
    Ks = lax.dynamic_slice_in_dim(K_ext, 2 * my, 2, axis=2)
    Vs = lax.dynamic_slice_in_dim(V_ext, 2 * my, 2, axis=2)
    Kh = jnp.transpose(Ks, (0, 2, 1, 3))
    Vh = jnp.transpose(Vs, (0, 2, 1, 3))

    def body(x_ref, wq_ref, wo_ref, k_ref, v_ref, out_ref,
             recv_buf, send_sem, recv_sems):
        my_pos = lax.axis_index("i")
        left = jnp.mod(my_pos - 1, N_DEV)
        right = jnp.mod(my_pos + 1, N_DEV)

        barrier_sem = pltpu.get_barrier_semaphore()
        for nbr in [left, right]:
            pl.semaphore_signal(
                barrier_sem, inc=1,
                device_id=(nbr,), device_id_type=pl.DeviceIdType.MESH,
            )
        pl.semaphore_wait(barrier_sem, 2)

        wq = wq_ref[...].astype(jnp.bfloat16)
        wo = wo_ref[...].astype(jnp.bfloat16)
        for b in range(B):
            xb = x_ref[b].astype(jnp.bfloat16)
            q = lax.dot(xb, wq, preferred_element_type=jnp.float32)
            cols = []
            for j in range(H_LOC):
                kv = j // 4
                qj = q[:, j * DH:(j + 1) * DH].astype(jnp.bfloat16)
                kj = k_ref[b, kv].astype(jnp.bfloat16)
                s = lax.dot_general(
                    qj, kj, (((1,), (1,)), ((), ())),
                    preferred_element_type=jnp.float32,
                ) * 0.125
                m = jnp.max(s, axis=1, keepdims=True)
                p = jnp.exp(s - m)
                l = jnp.sum(p, axis=1, keepdims=True)
                vj = v_ref[b, kv].astype(jnp.bfloat16)
                o = lax.dot(p.astype(jnp.bfloat16), vj,
                            preferred_element_type=jnp.float32)
                cols.append(o / l)
            attn = jnp.concatenate(cols, axis=1)
            out_ref[pl.ds(b * SQ, SQ), :] = lax.dot(
                attn.astype(jnp.bfloat16), wo,
                preferred_element_type=jnp.float32)

        for s in range(N_DEV - 1):
            c_send = jnp.mod(my_pos - s, N_DEV)
            c_recv = jnp.mod(my_pos - s - 1, N_DEV)
            rdma = pltpu.make_async_remote_copy(
                src_ref=out_ref.at[pl.ds(c_send * CHUNK, CHUNK), :],
                dst_ref=recv_buf.at[s],
                send_sem=send_sem,
                recv_sem=recv_sems.at[s],
                device_id=(right,),
                device_id_type=pl.DeviceIdType.MESH,
            )
            rdma.start()
            rdma.wait()
            out_ref[pl.ds(c_recv * CHUNK, CHUNK), :] = (
                out_ref[pl.ds(c_recv * CHUNK, CHUNK), :] + recv_buf[s]
            )

        for s in range(N_DEV - 1):
            c_send = jnp.mod(my_pos + 1 - s, N_DEV)
            rdma = pltpu.make_async_remote_copy(
                src_ref=out_ref.at[pl.ds(c_send * CHUNK, CHUNK), :],
                dst_ref=out_ref.at[pl.ds(c_send * CHUNK, CHUNK), :],
                send_sem=send_sem,
                recv_sem=recv_sems.at[s],
                device_id=(right,),
                device_id_type=pl.DeviceIdType.MESH,
            )
            rdma.start()
            rdma.wait()

    out2d = pl.pallas_call(
        body,
        out_shape=jax.ShapeDtypeStruct((N_ROWS, D_MODEL), jnp.float32),
        in_specs=[pl.BlockSpec(memory_space=pltpu.VMEM)] * 5,
        out_specs=pl.BlockSpec(memory_space=pltpu.VMEM),
        scratch_shapes=[
            pltpu.VMEM((N_DEV - 1, CHUNK, D_MODEL), jnp.float32),
            pltpu.SemaphoreType.DMA,
            pltpu.SemaphoreType.DMA((N_DEV - 1,)),
        ],
        compiler_params=pltpu.CompilerParams(collective_id=0),
    )(x, Wq, Wo, Kh, Vh)
    return out2d.reshape(B, SQ, D_MODEL)


# baseline (device time: 165163 ns/iter reference)
import jax
import jax.numpy as jnp
from jax import lax
from jax.experimental import pallas as pl
from jax.experimental.pallas import tpu as pltpu

N_DEV = 32
B, SQ, D_MODEL = 2, 256, 768
H_LOC = 8
DH = 64
N_ROWS = B * SQ
CHUNK = N_ROWS // N_DEV


def kernel(x, Wq, Wo, K_ext, V_ext):
    my = lax.axis_index("i")
    Ks = lax.dynamic_slice_in_dim(K_ext, 2 * my, 2, axis=2)
    Vs = lax.dynamic_slice_in_dim(V_ext, 2 * my, 2, axis=2)
    Kh = jnp.transpose(Ks, (0, 2, 1, 3))
    Vh = jnp.transpose(Vs, (0, 2, 1, 3))

    def body(x_ref, wq_ref, wo_ref, k_ref, v_ref, out_ref,
             recv_buf, send_sem, recv_sems):
        my_pos = lax.axis_index("i")
        left = jnp.mod(my_pos - 1, N_DEV)
        right = jnp.mod(my_pos + 1, N_DEV)

        barrier_sem = pltpu.get_barrier_semaphore()
        for nbr in [left, right]:
            pl.semaphore_signal(
                barrier_sem, inc=1,
                device_id=(nbr,), device_id_type=pl.DeviceIdType.MESH,
            )
        pl.semaphore_wait(barrier_sem, 2)

        wq = wq_ref[...].astype(jnp.bfloat16)
        wo = wo_ref[...].astype(jnp.bfloat16)
        for b in range(B):
            xb = x_ref[b].astype(jnp.bfloat16)
            q = lax.dot(xb, wq, preferred_element_type=jnp.float32)
            cols = []
            for j in range(H_LOC):
                kv = j // 4
                qj = q[:, j * DH:(j + 1) * DH].astype(jnp.bfloat16)
                kj = k_ref[b, kv].astype(jnp.bfloat16)
                s = lax.dot_general(
                    qj, kj, (((1,), (1,)), ((), ())),
                    preferred_element_type=jnp.float32,
                ) * 0.125
                m = jnp.max(s, axis=1, keepdims=True)
                p = jnp.exp(s - m)
                l = jnp.sum(p, axis=1, keepdims=True)
                vj = v_ref[b, kv].astype(jnp.bfloat16)
                o = lax.dot(p.astype(jnp.bfloat16), vj,
                            preferred_element_type=jnp.float32)
                cols.append(o / l)
            attn = jnp.concatenate(cols, axis=1)
            out_ref[pl.ds(b * SQ, SQ), :] = lax.dot(
                attn.astype(jnp.bfloat16), wo,
                preferred_element_type=jnp.float32)

        for s in range(N_DEV - 1):
            c_send = jnp.mod(my_pos - s, N_DEV)
            c_recv = jnp.mod(my_pos - s - 1, N_DEV)
            rdma = pltpu.make_async_remote_copy(
                src_ref=out_ref.at[pl.ds(c_send * CHUNK, CHUNK), :],
                dst_ref=recv_buf.at[s],
                send_sem=send_sem,
                recv_sem=recv_sems.at[s],
                device_id=(right,),
                device_id_type=pl.DeviceIdType.MESH,
            )
            rdma.start()
            rdma.wait()
            out_ref[pl.ds(c_recv * CHUNK, CHUNK), :] = (
                out_ref[pl.ds(c_recv * CHUNK, CHUNK), :] + recv_buf[s]
            )

        for s in range(N_DEV - 1):
            c_send = jnp.mod(my_pos + 1 - s, N_DEV)
            c_recv = jnp.mod(my_pos - s, N_DEV)
            slot = (N_DEV - 1) + s
            rdma = pltpu.make_async_remote_copy(
                src_ref=out_ref.at[pl.ds(c_send * CHUNK, CHUNK), :],
                dst_ref=recv_buf.at[slot],
                send_sem=send_sem,
                recv_sem=recv_sems.at[slot],
                device_id=(right,),
                device_id_type=pl.DeviceIdType.MESH,
            )
            rdma.start()
            rdma.wait()
            out_ref[pl.ds(c_recv * CHUNK, CHUNK), :] = recv_buf[slot]

    out2d = pl.pallas_call(
        body,
        out_shape=jax.ShapeDtypeStruct((N_ROWS, D_MODEL), jnp.float32),
        in_specs=[pl.BlockSpec(memory_space=pltpu.VMEM)] * 5,
        out_specs=pl.BlockSpec(memory_space=pltpu.VMEM),
        scratch_shapes=[
            pltpu.VMEM((2 * (N_DEV - 1), CHUNK, D_MODEL), jnp.float32),
            pltpu.SemaphoreType.DMA,
            pltpu.SemaphoreType.DMA((2 * (N_DEV - 1),)),
        ],
        compiler_params=pltpu.CompilerParams(collective_id=0),
    )(x, Wq, Wo, Kh, Vh)
    return out2d.reshape(B, SQ, D_MODEL)


# device time: 69521 ns/iter; 2.3757x vs baseline; 2.3757x over previous
import jax
import jax.numpy as jnp
from jax import lax
from jax.experimental import pallas as pl
from jax.experimental.pallas import tpu as pltpu

N_DEV = 32
B, SQ, D_MODEL = 2, 256, 768
H_LOC = 8
DH = 64
N_ROWS = B * SQ

STAGES_RS = [
    (1, "x", 256),
    (8, "z0", 128),
    (3, "y0", 64),
    (16, "z1", 32),
    (4, "y1", 16),
]
STAGES_AG = [(m, b, r) for (m, b, r) in reversed(STAGES_RS)]

_offs = []
_acc = 0
for _m, _b, _r in STAGES_RS + STAGES_AG:
    _offs.append(_acc)
    _acc += _r
RS_OFF = _offs[:5]
AG_OFF = _offs[5:]
SCRATCH_ROWS = _acc


def kernel(x, Wq, Wo, K_ext, V_ext):
    my = lax.axis_index("i")
    Ks = lax.dynamic_slice_in_dim(K_ext, 2 * my, 2, axis=2)
    Vs = lax.dynamic_slice_in_dim(V_ext, 2 * my, 2, axis=2)
    Kh = jnp.transpose(Ks, (0, 2, 1, 3))
    Vh = jnp.transpose(Vs, (0, 2, 1, 3))

    def body(x_ref, wq_ref, wo_ref, k_ref, v_ref, out_ref,
             recv_buf, send_sem, recv_sems):
        p = lax.axis_index("i")
        q_plane = jnp.mod(p, 8)
        z = p // 8
        y = q_plane // 2
        r = jnp.mod(q_plane, 2)
        xb_coord = jnp.where(jnp.mod(y, 2) == 0, r, 1 - r)
        bits = {
            "x": xb_coord,
            "y0": jnp.mod(y, 2),
            "y1": y // 2,
            "z0": jnp.mod(z, 2),
            "z1": z // 2,
        }

        barrier_sem = pltpu.get_barrier_semaphore()
        for mask, _, _ in STAGES_RS:
            pl.semaphore_signal(
                barrier_sem, inc=1,
                device_id=(jnp.bitwise_xor(p, mask),),
                device_id_type=pl.DeviceIdType.MESH,
            )
        pl.semaphore_wait(barrier_sem, len(STAGES_RS))

        wq = wq_ref[...].astype(jnp.bfloat16)
        wo = wo_ref[...].astype(jnp.bfloat16)
        for b in range(B):
            xb = x_ref[b].astype(jnp.bfloat16)
            qm = lax.dot(xb, wq, preferred_element_type=jnp.float32)
            cols = []
            for j in range(H_LOC):
                kv = j // 4
                qj = qm[:, j * DH:(j + 1) * DH].astype(jnp.bfloat16)
                kj = k_ref[b, kv].astype(jnp.bfloat16)
                s = lax.dot_general(
                    qj, kj, (((1,), (1,)), ((), ())),
                    preferred_element_type=jnp.float32,
                ) * 0.125
                m = jnp.max(s, axis=1, keepdims=True)
                pj = jnp.exp(s - m)
                l = jnp.sum(pj, axis=1, keepdims=True)
                vj = v_ref[b, kv].astype(jnp.bfloat16)
                o = lax.dot(pj.astype(jnp.bfloat16), vj,
                            preferred_element_type=jnp.float32)
                cols.append(o / l)
            attn = jnp.concatenate(cols, axis=1)
            out_ref[pl.ds(b * SQ, SQ), :] = lax.dot(
                attn.astype(jnp.bfloat16), wo,
                preferred_element_type=jnp.float32)

        lo = jnp.int32(0)
        for k, (mask, bit_name, half) in enumerate(STAGES_RS):
            bit = bits[bit_name]
            partner = jnp.bitwise_xor(p, mask)
            keep_lo = lo + bit * half
            send_lo = lo + (1 - bit) * half
            rdma = pltpu.make_async_remote_copy(
                src_ref=out_ref.at[pl.ds(send_lo, half), :],
                dst_ref=recv_buf.at[pl.ds(RS_OFF[k], half), :],
                send_sem=send_sem,
                recv_sem=recv_sems.at[k],
                device_id=(partner,),
                device_id_type=pl.DeviceIdType.MESH,
            )
            rdma.start()
            rdma.wait()
            out_ref[pl.ds(keep_lo, half), :] = (
                out_ref[pl.ds(keep_lo, half), :]
                + recv_buf[pl.ds(RS_OFF[k], half), :]
            )
            lo = keep_lo

        for k, (mask, bit_name, sz) in enumerate(STAGES_AG):
            bit = bits[bit_name]
            partner = jnp.bitwise_xor(p, mask)
            partner_lo = lo + (1 - 2 * bit) * sz
            rdma = pltpu.make_async_remote_copy(
                src_ref=out_ref.at[pl.ds(lo, sz), :],
                dst_ref=recv_buf.at[pl.ds(AG_OFF[k], sz), :],
                send_sem=send_sem,
                recv_sem=recv_sems.at[len(STAGES_RS) + k],
                device_id=(partner,),
                device_id_type=pl.DeviceIdType.MESH,
            )
            rdma.start()
            rdma.wait()
            out_ref[pl.ds(partner_lo, sz), :] = recv_buf[
                pl.ds(AG_OFF[k], sz), :
            ]
            lo = lo - bit * sz

    out2d = pl.pallas_call(
        body,
        out_shape=jax.ShapeDtypeStruct((N_ROWS, D_MODEL), jnp.float32),
        in_specs=[pl.BlockSpec(memory_space=pltpu.VMEM)] * 5,
        out_specs=pl.BlockSpec(memory_space=pltpu.VMEM),
        scratch_shapes=[
            pltpu.VMEM((SCRATCH_ROWS, D_MODEL), jnp.float32),
            pltpu.SemaphoreType.DMA,
            pltpu.SemaphoreType.DMA((10,)),
        ],
        compiler_params=pltpu.CompilerParams(collective_id=0),
    )(x, Wq, Wo, Kh, Vh)
    return out2d.reshape(B, SQ, D_MODEL)


# device time: 48072 ns/iter; 3.4357x vs baseline; 1.4462x over previous
import jax
import jax.numpy as jnp
from jax import lax
from jax.experimental import pallas as pl
from jax.experimental.pallas import tpu as pltpu

N_DEV = 32
B, SQ, D_MODEL = 2, 256, 768
H_LOC = 8
DH = 64
N_ROWS = B * SQ

STAGES_RS = [
    (1, "x", 256),
    (8, "z0", 128),
    (3, "y0", 64),
    (16, "z1", 32),
    (4, "y1", 16),
]
STAGES_AG = [(m, b, r) for (m, b, r) in reversed(STAGES_RS)]

_offs = []
_acc = 0
for _m, _b, _r in STAGES_RS + STAGES_AG:
    _offs.append(_acc)
    _acc += _r
RS_OFF = _offs[:5]
AG_OFF = _offs[5:]
SCRATCH_ROWS = _acc


def kernel(x, Wq, Wo, K_ext, V_ext):
    my = lax.axis_index("i")
    Ks = lax.dynamic_slice_in_dim(K_ext, 2 * my, 2, axis=2)
    Vs = lax.dynamic_slice_in_dim(V_ext, 2 * my, 2, axis=2)
    Kh = jnp.transpose(Ks, (0, 2, 1, 3))
    Vh = jnp.transpose(Vs, (0, 2, 1, 3))

    def body(x_ref, wq_ref, wo_ref, k_ref, v_ref, out_ref,
             send_buf, recv_buf, send_sem, recv_sems):
        p = lax.axis_index("i")
        q_plane = jnp.mod(p, 8)
        z = p // 8
        y = q_plane // 2
        r = jnp.mod(q_plane, 2)
        xb_coord = jnp.where(jnp.mod(y, 2) == 0, r, 1 - r)
        bits = {
            "x": xb_coord,
            "y0": jnp.mod(y, 2),
            "y1": y // 2,
            "z0": jnp.mod(z, 2),
            "z1": z // 2,
        }

        barrier_sem = pltpu.get_barrier_semaphore()
        for mask, _, _ in STAGES_RS:
            pl.semaphore_signal(
                barrier_sem, inc=1,
                device_id=(jnp.bitwise_xor(p, mask),),
                device_id_type=pl.DeviceIdType.MESH,
            )
        pl.semaphore_wait(barrier_sem, len(STAGES_RS))

        wq = wq_ref[...].astype(jnp.bfloat16)
        wo = wo_ref[...].astype(jnp.bfloat16)

        def compute_batch(b_eff):
            xb = x_ref[b_eff].astype(jnp.bfloat16)
            qm = lax.dot(xb, wq, preferred_element_type=jnp.float32)
            cols = []
            for j in range(H_LOC):
                kv = j // 4
                qj = qm[:, j * DH:(j + 1) * DH].astype(jnp.bfloat16)
                kj = k_ref[b_eff, kv].astype(jnp.bfloat16)
                s = lax.dot_general(
                    qj, kj, (((1,), (1,)), ((), ())),
                    preferred_element_type=jnp.float32,
                ) * 0.125
                m = jnp.max(s, axis=1, keepdims=True)
                pj = jnp.exp(s - m)
                l = jnp.sum(pj, axis=1, keepdims=True)
                vj = v_ref[b_eff, kv].astype(jnp.bfloat16)
                o = lax.dot(pj.astype(jnp.bfloat16), vj,
                            preferred_element_type=jnp.float32)
                cols.append(o / l)
            attn = jnp.concatenate(cols, axis=1)
            out_ref[pl.ds(b_eff * SQ, SQ), :] = lax.dot(
                attn.astype(jnp.bfloat16), wo,
                preferred_element_type=jnp.float32)

        def start_exchange(k, src_lo, rows, partner, slot_off):
            send_buf[pl.ds(slot_off, rows), :] = out_ref[
                pl.ds(src_lo, rows), :
            ].astype(jnp.bfloat16)
            rdma = pltpu.make_async_remote_copy(
                src_ref=send_buf.at[pl.ds(slot_off, rows), :],
                dst_ref=recv_buf.at[pl.ds(slot_off, rows), :],
                send_sem=send_sem,
                recv_sem=recv_sems.at[k],
                device_id=(partner,),
                device_id_type=pl.DeviceIdType.MESH,
            )
            rdma.start()
            return rdma

        xbit = bits["x"]
        mask0, _, half0 = STAGES_RS[0]
        compute_batch(1 - xbit)
        rdma0 = start_exchange(
            0, (1 - xbit) * half0, half0, jnp.bitwise_xor(p, mask0), RS_OFF[0]
        )
        compute_batch(xbit)
        rdma0.wait()
        keep_lo = xbit * half0
        out_ref[pl.ds(keep_lo, half0), :] = (
            out_ref[pl.ds(keep_lo, half0), :]
            + recv_buf[pl.ds(RS_OFF[0], half0), :].astype(jnp.float32)
        )
        lo = keep_lo

        for k, (mask, bit_name, half) in enumerate(STAGES_RS):
            if k == 0:
                continue
            bit = bits[bit_name]
            keep_lo = lo + bit * half
            send_lo = lo + (1 - bit) * half
            rdma = start_exchange(
                k, send_lo, half, jnp.bitwise_xor(p, mask), RS_OFF[k]
            )
            rdma.wait()
            out_ref[pl.ds(keep_lo, half), :] = (
                out_ref[pl.ds(keep_lo, half), :]
                + recv_buf[pl.ds(RS_OFF[k], half), :].astype(jnp.float32)
            )
            lo = keep_lo

        for k, (mask, bit_name, sz) in enumerate(STAGES_AG):
            bit = bits[bit_name]
            partner_lo = lo + (1 - 2 * bit) * sz
            rdma = start_exchange(
                len(STAGES_RS) + k, lo, sz, jnp.bitwise_xor(p, mask), AG_OFF[k]
            )
            rdma.wait()
            out_ref[pl.ds(partner_lo, sz), :] = recv_buf[
                pl.ds(AG_OFF[k], sz), :
            ].astype(jnp.float32)
            lo = lo - bit * sz

    out2d = pl.pallas_call(
        body,
        out_shape=jax.ShapeDtypeStruct((N_ROWS, D_MODEL), jnp.float32),
        in_specs=[pl.BlockSpec(memory_space=pltpu.VMEM)] * 5,
        out_specs=pl.BlockSpec(memory_space=pltpu.VMEM),
        scratch_shapes=[
            pltpu.VMEM((SCRATCH_ROWS, D_MODEL), jnp.bfloat16),
            pltpu.VMEM((SCRATCH_ROWS, D_MODEL), jnp.bfloat16),
            pltpu.SemaphoreType.DMA,
            pltpu.SemaphoreType.DMA((10,)),
        ],
        compiler_params=pltpu.CompilerParams(collective_id=0),
    )(x, Wq, Wo, Kh, Vh)
    return out2d.reshape(B, SQ, D_MODEL)


# device time: 44762 ns/iter; 3.6898x vs baseline; 1.0739x over previous
import jax
import jax.numpy as jnp
from jax import lax
from jax.experimental import pallas as pl
from jax.experimental.pallas import tpu as pltpu

N_DEV = 32
B, SQ, D_MODEL = 2, 256, 768
H_LOC = 8
DH = 64
N_ROWS = B * SQ

Y_MASK = {1: 3, 2: 4, 3: 7}
Z_MASK = {1: 8, 2: 16, 3: 24}

SLOT_ROWS = [256, 64, 64, 64, 16, 16, 16, 16, 16, 16, 64, 64, 64, 256]
SLOT_OFF = []
_acc = 0
for _r in SLOT_ROWS:
    SLOT_OFF.append(_acc)
    _acc += _r
SCRATCH_ROWS = _acc
N_EXCH = len(SLOT_ROWS)


def kernel(x, Wq, Wo, K_ext, V_ext):
    my = lax.axis_index("i")
    Ks = lax.dynamic_slice_in_dim(K_ext, 2 * my, 2, axis=2)
    Vs = lax.dynamic_slice_in_dim(V_ext, 2 * my, 2, axis=2)
    Kh = jnp.transpose(Ks, (0, 2, 1, 3))
    Vh = jnp.transpose(Vs, (0, 2, 1, 3))

    def body(x_ref, wq_ref, wo_ref, k_ref, v_ref, out_ref,
             send_buf, recv_buf, send_sems, recv_sems):
        p = lax.axis_index("i")
        q_plane = jnp.mod(p, 8)
        z = p // 8
        y = q_plane // 2
        r = jnp.mod(q_plane, 2)
        xbit = jnp.where(jnp.mod(y, 2) == 0, r, 1 - r)

        barrier_sem = pltpu.get_barrier_semaphore()
        partner_masks = [1] + [Z_MASK[d] for d in (1, 2, 3)] + [
            Y_MASK[d] for d in (1, 2, 3)
        ]
        for mask in partner_masks:
            pl.semaphore_signal(
                barrier_sem, inc=1,
                device_id=(jnp.bitwise_xor(p, mask),),
                device_id_type=pl.DeviceIdType.MESH,
            )
        pl.semaphore_wait(barrier_sem, len(partner_masks))

        wq = wq_ref[...].astype(jnp.bfloat16)
        wo = wo_ref[...].astype(jnp.bfloat16)

        def compute_batch(b_eff):
            xb = x_ref[b_eff].astype(jnp.bfloat16)
            qm = lax.dot(xb, wq, preferred_element_type=jnp.float32)
            cols = []
            for j in range(H_LOC):
                kv = j // 4
                qj = qm[:, j * DH:(j + 1) * DH].astype(jnp.bfloat16)
                kj = k_ref[b_eff, kv].astype(jnp.bfloat16)
                s = lax.dot_general(
                    qj, kj, (((1,), (1,)), ((), ())),
                    preferred_element_type=jnp.float32,
                ) * 0.125
                m = jnp.max(s, axis=1, keepdims=True)
                pj = jnp.exp(s - m)
                l = jnp.sum(pj, axis=1, keepdims=True)
                vj = v_ref[b_eff, kv].astype(jnp.bfloat16)
                o = lax.dot(pj.astype(jnp.bfloat16), vj,
                            preferred_element_type=jnp.float32)
                cols.append(o / l)
            attn = jnp.concatenate(cols, axis=1)
            out_ref[pl.ds(b_eff * SQ, SQ), :] = lax.dot(
                attn.astype(jnp.bfloat16), wo,
                preferred_element_type=jnp.float32)

        def start_exchange(idx, src_lo, rows, partner, stage=True):
            off = SLOT_OFF[idx]
            if stage:
                send_buf[pl.ds(off, rows), :] = out_ref[
                    pl.ds(src_lo, rows), :
                ].astype(jnp.bfloat16)
                src = send_buf.at[pl.ds(off, rows), :]
            else:
                src = send_buf.at[pl.ds(src_lo, rows), :]
            rdma = pltpu.make_async_remote_copy(
                src_ref=src,
                dst_ref=recv_buf.at[pl.ds(off, rows), :],
                send_sem=send_sems.at[idx],
                recv_sem=recv_sems.at[idx],
                device_id=(partner,),
                device_id_type=pl.DeviceIdType.MESH,
            )
            rdma.start()
            return rdma

        compute_batch(1 - xbit)
        rdma0 = start_exchange(
            0, (1 - xbit) * 256, 256, jnp.bitwise_xor(p, 1)
        )
        compute_batch(xbit)
        rdma0.wait()
        lo = xbit * 256
        out_ref[pl.ds(lo, 256), :] = (
            out_ref[pl.ds(lo, 256), :]
            + recv_buf[pl.ds(SLOT_OFF[0], 256), :].astype(jnp.float32)
        )

        for coord, masks, u, idx0 in ((z, Z_MASK, 64, 1), (y, Y_MASK, 16, 4)):
            rdmas = []
            for d in (1, 2, 3):
                cd = jnp.bitwise_xor(coord, d)
                rdmas.append(start_exchange(
                    idx0 + d - 1, lo + cd * u, u,
                    jnp.bitwise_xor(p, masks[d]),
                ))
            for rdma in rdmas:
                rdma.wait()
            keep_lo = lo + coord * u
            out_ref[pl.ds(keep_lo, u), :] = (
                out_ref[pl.ds(keep_lo, u), :]
                + recv_buf[pl.ds(SLOT_OFF[idx0], u), :].astype(jnp.float32)
                + recv_buf[pl.ds(SLOT_OFF[idx0 + 1], u), :].astype(jnp.float32)
                + recv_buf[pl.ds(SLOT_OFF[idx0 + 2], u), :].astype(jnp.float32)
            )
            lo = keep_lo

        for coord, masks, u, idx0 in ((y, Y_MASK, 16, 7), (z, Z_MASK, 64, 10)):
            off0 = SLOT_OFF[idx0]
            send_buf[pl.ds(off0, u), :] = out_ref[
                pl.ds(lo, u), :
            ].astype(jnp.bfloat16)
            rdmas = []
            for d in (1, 2, 3):
                rdmas.append(start_exchange(
                    idx0 + d - 1, off0, u,
                    jnp.bitwise_xor(p, masks[d]), stage=False,
                ))
            for rdma in rdmas:
                rdma.wait()
            base = lo - coord * u
            for d in (1, 2, 3):
                cd = jnp.bitwise_xor(coord, d)
                out_ref[pl.ds(base + cd * u, u), :] = recv_buf[
                    pl.ds(SLOT_OFF[idx0 + d - 1], u), :
                ].astype(jnp.float32)
            lo = base

        rdma_x = start_exchange(13, lo, 256, jnp.bitwise_xor(p, 1))
        rdma_x.wait()
        partner_lo = (1 - xbit) * 256
        out_ref[pl.ds(partner_lo, 256), :] = recv_buf[
            pl.ds(SLOT_OFF[13], 256), :
        ].astype(jnp.float32)

    out2d = pl.pallas_call(
        body,
        out_shape=jax.ShapeDtypeStruct((N_ROWS, D_MODEL), jnp.float32),
        in_specs=[pl.BlockSpec(memory_space=pltpu.VMEM)] * 5,
        out_specs=pl.BlockSpec(memory_space=pltpu.VMEM),
        scratch_shapes=[
            pltpu.VMEM((SCRATCH_ROWS, D_MODEL), jnp.bfloat16),
            pltpu.VMEM((SCRATCH_ROWS, D_MODEL), jnp.bfloat16),
            pltpu.SemaphoreType.DMA((N_EXCH,)),
            pltpu.SemaphoreType.DMA((N_EXCH,)),
        ],
        compiler_params=pltpu.CompilerParams(collective_id=0),
    )(x, Wq, Wo, Kh, Vh)
    return out2d.reshape(B, SQ, D_MODEL)


# device time: 41758 ns/iter; 3.9552x vs baseline; 1.0719x over previous
import jax
import jax.numpy as jnp
from jax import lax
from jax.experimental import pallas as pl
from jax.experimental.pallas import tpu as pltpu

N_DEV = 32
B, SQ, D_MODEL = 2, 256, 768
H_LOC = 8
DH = 64
N_ROWS = B * SQ

Y_MASK = {1: 3, 2: 4, 3: 7}
Z_MASK = {1: 8, 2: 16, 3: 24}

SLOT_ROWS = [256, 64, 64, 64, 16, 16, 16, 16, 16, 16, 64, 64, 64,
             64, 64, 64, 64]
SLOT_OFF = []
_acc = 0
for _r in SLOT_ROWS:
    SLOT_OFF.append(_acc)
    _acc += _r
SCRATCH_ROWS = _acc
N_EXCH = len(SLOT_ROWS)


def kernel(x, Wq, Wo, K_ext, V_ext):
    my = lax.axis_index("i")
    Ks = lax.dynamic_slice_in_dim(K_ext, 2 * my, 2, axis=2)
    Vs = lax.dynamic_slice_in_dim(V_ext, 2 * my, 2, axis=2)
    Kh = jnp.transpose(Ks, (0, 2, 1, 3))
    Vh = jnp.transpose(Vs, (0, 2, 1, 3))

    def body(x_ref, wq_ref, wo_ref, k_ref, v_ref, out_ref,
             send_buf, recv_buf, send_sems, recv_sems):
        p = lax.axis_index("i")
        q_plane = jnp.mod(p, 8)
        z = p // 8
        y = q_plane // 2
        r = jnp.mod(q_plane, 2)
        xbit = jnp.where(jnp.mod(y, 2) == 0, r, 1 - r)

        barrier_sem = pltpu.get_barrier_semaphore()
        partner_masks = [1] + [Z_MASK[d] for d in (1, 2, 3)] + [
            Y_MASK[d] for d in (1, 2, 3)
        ]
        for mask in partner_masks:
            pl.semaphore_signal(
                barrier_sem, inc=1,
                device_id=(jnp.bitwise_xor(p, mask),),
                device_id_type=pl.DeviceIdType.MESH,
            )
        pl.semaphore_wait(barrier_sem, len(partner_masks))

        wq = wq_ref[...].astype(jnp.bfloat16)
        wo = wo_ref[...].astype(jnp.bfloat16)

        def compute_batch(b_eff):
            xb = x_ref[b_eff].astype(jnp.bfloat16)
            qm = lax.dot(xb, wq, preferred_element_type=jnp.float32)
            cols = []
            for j in range(H_LOC):
                kv = j // 4
                qj = qm[:, j * DH:(j + 1) * DH].astype(jnp.bfloat16)
                kj = k_ref[b_eff, kv].astype(jnp.bfloat16)
                s = lax.dot_general(
                    qj, kj, (((1,), (1,)), ((), ())),
                    preferred_element_type=jnp.float32,
                ) * 0.125
                m = jnp.max(s, axis=1, keepdims=True)
                pj = jnp.exp(s - m)
                l = jnp.sum(pj, axis=1, keepdims=True)
                vj = v_ref[b_eff, kv].astype(jnp.bfloat16)
                o = lax.dot(pj.astype(jnp.bfloat16), vj,
                            preferred_element_type=jnp.float32)
                cols.append(o / l)
            attn = jnp.concatenate(cols, axis=1)
            out_ref[pl.ds(b_eff * SQ, SQ), :] = lax.dot(
                attn.astype(jnp.bfloat16), wo,
                preferred_element_type=jnp.float32)

        def start_exchange(idx, src_lo, rows, partner, stage=True,
                           src_buf=None):
            off = SLOT_OFF[idx]
            if stage:
                send_buf[pl.ds(off, rows), :] = out_ref[
                    pl.ds(src_lo, rows), :
                ].astype(jnp.bfloat16)
                src = send_buf.at[pl.ds(off, rows), :]
            else:
                buf = send_buf if src_buf is None else src_buf
                src = buf.at[pl.ds(src_lo, rows), :]
            rdma = pltpu.make_async_remote_copy(
                src_ref=src,
                dst_ref=recv_buf.at[pl.ds(off, rows), :],
                send_sem=send_sems.at[idx],
                recv_sem=recv_sems.at[idx],
                device_id=(partner,),
                device_id_type=pl.DeviceIdType.MESH,
            )
            rdma.start()
            return rdma

        compute_batch(1 - xbit)
        rdma0 = start_exchange(
            0, (1 - xbit) * 256, 256, jnp.bitwise_xor(p, 1)
        )
        compute_batch(xbit)
        rdma0.wait()
        lo = xbit * 256
        out_ref[pl.ds(lo, 256), :] = (
            out_ref[pl.ds(lo, 256), :]
            + recv_buf[pl.ds(SLOT_OFF[0], 256), :].astype(jnp.float32)
        )

        for coord, masks, u, idx0 in ((z, Z_MASK, 64, 1), (y, Y_MASK, 16, 4)):
            rdmas = []
            for d in (1, 2, 3):
                cd = jnp.bitwise_xor(coord, d)
                rdmas.append(start_exchange(
                    idx0 + d - 1, lo + cd * u, u,
                    jnp.bitwise_xor(p, masks[d]),
                ))
            for rdma in rdmas:
                rdma.wait()
            keep_lo = lo + coord * u
            out_ref[pl.ds(keep_lo, u), :] = (
                out_ref[pl.ds(keep_lo, u), :]
                + recv_buf[pl.ds(SLOT_OFF[idx0], u), :].astype(jnp.float32)
                + recv_buf[pl.ds(SLOT_OFF[idx0 + 1], u), :].astype(jnp.float32)
                + recv_buf[pl.ds(SLOT_OFF[idx0 + 2], u), :].astype(jnp.float32)
            )
            lo = keep_lo

        off7 = SLOT_OFF[7]
        send_buf[pl.ds(off7, 16), :] = out_ref[pl.ds(lo, 16), :].astype(
            jnp.bfloat16)
        rdmas = []
        for d in (1, 2, 3):
            rdmas.append(start_exchange(
                7 + d - 1, off7, 16,
                jnp.bitwise_xor(p, Y_MASK[d]), stage=False,
            ))
        for rdma in rdmas:
            rdma.wait()
        base = lo - y * 16
        for d in (1, 2, 3):
            cd = jnp.bitwise_xor(y, d)
            out_ref[pl.ds(base + cd * 16, 16), :] = recv_buf[
                pl.ds(SLOT_OFF[7 + d - 1], 16), :
            ].astype(jnp.float32)
        lo = base

        x_partner = jnp.bitwise_xor(p, 1)
        rdma_x = [start_exchange(13, lo, 64, x_partner)]
        off10 = SLOT_OFF[10]
        send_buf[pl.ds(off10, 64), :] = out_ref[pl.ds(lo, 64), :].astype(
            jnp.bfloat16)
        rdma_z = []
        for d in (1, 2, 3):
            rdma_z.append(start_exchange(
                10 + d - 1, off10, 64,
                jnp.bitwise_xor(p, Z_MASK[d]), stage=False,
            ))
        base = lo - z * 64
        for d in (1, 2, 3):
            rdma_z[d - 1].wait()
            cd = jnp.bitwise_xor(z, d)
            out_ref[pl.ds(base + cd * 64, 64), :] = recv_buf[
                pl.ds(SLOT_OFF[10 + d - 1], 64), :
            ].astype(jnp.float32)
            rdma_x.append(start_exchange(
                13 + d, SLOT_OFF[10 + d - 1], 64, x_partner, stage=False,
                src_buf=recv_buf,
            ))
        partner_base = (1 - xbit) * 256
        for d in (0, 1, 2, 3):
            rdma_x[d].wait()
            cd = jnp.bitwise_xor(z, d)
            out_ref[pl.ds(partner_base + cd * 64, 64), :] = recv_buf[
                pl.ds(SLOT_OFF[13 + d], 64), :
            ].astype(jnp.float32)

    out2d = pl.pallas_call(
        body,
        out_shape=jax.ShapeDtypeStruct((N_ROWS, D_MODEL), jnp.float32),
        in_specs=[pl.BlockSpec(memory_space=pltpu.VMEM)] * 5,
        out_specs=pl.BlockSpec(memory_space=pltpu.VMEM),
        scratch_shapes=[
            pltpu.VMEM((SCRATCH_ROWS, D_MODEL), jnp.bfloat16),
            pltpu.VMEM((SCRATCH_ROWS, D_MODEL), jnp.bfloat16),
            pltpu.SemaphoreType.DMA((N_EXCH,)),
            pltpu.SemaphoreType.DMA((N_EXCH,)),
        ],
        compiler_params=pltpu.CompilerParams(collective_id=0),
    )(x, Wq, Wo, Kh, Vh)
    return out2d.reshape(B, SQ, D_MODEL)


# device time: 41630 ns/iter; 3.9674x vs baseline; 1.0031x over previous
import jax
import jax.numpy as jnp
from jax import lax
from jax.experimental import pallas as pl
from jax.experimental.pallas import tpu as pltpu

N_DEV = 32
B, SQ, D_MODEL = 2, 256, 768
H_LOC = 8
DH = 64
N_ROWS = B * SQ

Y_MASK = {1: 3, 2: 4, 3: 7}
Z_MASK = {1: 8, 2: 16, 3: 24}

SLOT_ROWS = [16, 64, 64, 64, 16, 16, 16, 16, 16, 16, 64, 64, 64,
             64, 64, 64, 64, 64, 64, 64, 64]
SLOT_OFF = []
_acc = 0
for _r in SLOT_ROWS:
    SLOT_OFF.append(_acc)
    _acc += _r
SCRATCH_ROWS = _acc
N_EXCH = len(SLOT_ROWS)


def kernel(x, Wq, Wo, K_ext, V_ext):
    my = lax.axis_index("i")
    Ks = lax.dynamic_slice_in_dim(K_ext, 2 * my, 2, axis=2)
    Vs = lax.dynamic_slice_in_dim(V_ext, 2 * my, 2, axis=2)
    Kh = jnp.transpose(Ks, (0, 2, 1, 3))
    Vh = jnp.transpose(Vs, (0, 2, 1, 3))

    def body(x_ref, wq_ref, wo_ref, k_ref, v_ref, out_ref,
             send_buf, recv_buf, send_sems, recv_sems):
        p = lax.axis_index("i")
        q_plane = jnp.mod(p, 8)
        z = p // 8
        y = q_plane // 2
        r = jnp.mod(q_plane, 2)
        xbit = jnp.where(jnp.mod(y, 2) == 0, r, 1 - r)

        barrier_sem = pltpu.get_barrier_semaphore()
        partner_masks = [1] + [Z_MASK[d] for d in (1, 2, 3)] + [
            Y_MASK[d] for d in (1, 2, 3)
        ]
        for mask in partner_masks:
            pl.semaphore_signal(
                barrier_sem, inc=1,
                device_id=(jnp.bitwise_xor(p, mask),),
                device_id_type=pl.DeviceIdType.MESH,
            )
        pl.semaphore_wait(barrier_sem, len(partner_masks))

        wq = wq_ref[...].astype(jnp.bfloat16)
        wo = wo_ref[...].astype(jnp.bfloat16)

        def compute_batch(b_eff):
            xb = x_ref[b_eff].astype(jnp.bfloat16)
            qm = lax.dot(xb, wq, preferred_element_type=jnp.float32)
            cols = []
            for j in range(H_LOC):
                kv = j // 4
                qj = qm[:, j * DH:(j + 1) * DH].astype(jnp.bfloat16)
                kj = k_ref[b_eff, kv].astype(jnp.bfloat16)
                s = lax.dot_general(
                    qj, kj, (((1,), (1,)), ((), ())),
                    preferred_element_type=jnp.float32,
                ) * 0.125
                m = jnp.max(s, axis=1, keepdims=True)
                pj = jnp.exp(s - m)
                l = jnp.sum(pj, axis=1, keepdims=True)
                vj = v_ref[b_eff, kv].astype(jnp.bfloat16)
                o = lax.dot(pj.astype(jnp.bfloat16), vj,
                            preferred_element_type=jnp.float32)
                cols.append(o / l)
            attn = jnp.concatenate(cols, axis=1)
            out_ref[pl.ds(b_eff * SQ, SQ), :] = lax.dot(
                attn.astype(jnp.bfloat16), wo,
                preferred_element_type=jnp.float32)

        def start_exchange(idx, src_lo, rows, partner, stage=True,
                           src_buf=None):
            off = SLOT_OFF[idx]
            if stage:
                send_buf[pl.ds(off, rows), :] = out_ref[
                    pl.ds(src_lo, rows), :
                ].astype(jnp.bfloat16)
                src = send_buf.at[pl.ds(off, rows), :]
            else:
                buf = send_buf if src_buf is None else src_buf
                src = buf.at[pl.ds(src_lo, rows), :]
            rdma = pltpu.make_async_remote_copy(
                src_ref=src,
                dst_ref=recv_buf.at[pl.ds(off, rows), :],
                send_sem=send_sems.at[idx],
                recv_sem=recv_sems.at[idx],
                device_id=(partner,),
                device_id_type=pl.DeviceIdType.MESH,
            )
            rdma.start()
            return rdma

        x_partner = jnp.bitwise_xor(p, 1)
        compute_batch(1 - xbit)
        send_half = (1 - xbit) * 256
        rdma_xrs = [
            start_exchange(17 + j, send_half + j * 64, 64, x_partner)
            for j in range(4)
        ]
        compute_batch(xbit)
        lo = xbit * 256

        z_rdmas = []
        for d in (1, 2, 3):
            off = SLOT_OFF[d]
            rdma = pltpu.make_async_remote_copy(
                src_ref=send_buf.at[pl.ds(off, 64), :],
                dst_ref=recv_buf.at[pl.ds(off, 64), :],
                send_sem=send_sems.at[d],
                recv_sem=recv_sems.at[d],
                device_id=(jnp.bitwise_xor(p, Z_MASK[d]),),
                device_id_type=pl.DeviceIdType.MESH,
            )
            z_rdmas.append((rdma, jnp.bitwise_xor(z, d), off))

        for j in range(4):
            rdma_xrs[j].wait()
            out_ref[pl.ds(lo + j * 64, 64), :] = (
                out_ref[pl.ds(lo + j * 64, 64), :]
                + recv_buf[pl.ds(SLOT_OFF[17 + j], 64), :].astype(jnp.float32)
            )
            for rdma, cd, off in z_rdmas:
                @pl.when(jnp.equal(cd, j))
                def _(rdma=rdma, cd=cd, off=off):
                    send_buf[pl.ds(off, 64), :] = out_ref[
                        pl.ds(lo + cd * 64, 64), :
                    ].astype(jnp.bfloat16)
                    rdma.start()

        keep_lo = lo + z * 64
        for d in (1, 2, 3):
            z_rdmas[d - 1][0].wait()
            out_ref[pl.ds(keep_lo, 64), :] = (
                out_ref[pl.ds(keep_lo, 64), :]
                + recv_buf[pl.ds(SLOT_OFF[d], 64), :].astype(jnp.float32)
            )
        lo = keep_lo

        rdmas = []
        for d in (1, 2, 3):
            cd = jnp.bitwise_xor(y, d)
            rdmas.append(start_exchange(
                4 + d - 1, lo + cd * 16, 16,
                jnp.bitwise_xor(p, Y_MASK[d]),
            ))
        for rdma in rdmas:
            rdma.wait()
        keep_lo = lo + y * 16
        out_ref[pl.ds(keep_lo, 16), :] = (
            out_ref[pl.ds(keep_lo, 16), :]
            + recv_buf[pl.ds(SLOT_OFF[4], 16), :].astype(jnp.float32)
            + recv_buf[pl.ds(SLOT_OFF[5], 16), :].astype(jnp.float32)
            + recv_buf[pl.ds(SLOT_OFF[6], 16), :].astype(jnp.float32)
        )
        lo = keep_lo

        off7 = SLOT_OFF[7]
        send_buf[pl.ds(off7, 16), :] = out_ref[pl.ds(lo, 16), :].astype(
            jnp.bfloat16)
        rdmas = []
        for d in (1, 2, 3):
            rdmas.append(start_exchange(
                7 + d - 1, off7, 16,
                jnp.bitwise_xor(p, Y_MASK[d]), stage=False,
            ))
        for rdma in rdmas:
            rdma.wait()
        base = lo - y * 16
        for d in (1, 2, 3):
            cd = jnp.bitwise_xor(y, d)
            out_ref[pl.ds(base + cd * 16, 16), :] = recv_buf[
                pl.ds(SLOT_OFF[7 + d - 1], 16), :
            ].astype(jnp.float32)
        lo = base

        x_partner = jnp.bitwise_xor(p, 1)
        rdma_x = [start_exchange(13, lo, 64, x_partner)]
        off10 = SLOT_OFF[10]
        send_buf[pl.ds(off10, 64), :] = out_ref[pl.ds(lo, 64), :].astype(
            jnp.bfloat16)
        rdma_z = []
        for d in (1, 2, 3):
            rdma_z.append(start_exchange(
                10 + d - 1, off10, 64,
                jnp.bitwise_xor(p, Z_MASK[d]), stage=False,
            ))
        base = lo - z * 64
        for d in (1, 2, 3):
            rdma_z[d - 1].wait()
            cd = jnp.bitwise_xor(z, d)
            out_ref[pl.ds(base + cd * 64, 64), :] = recv_buf[
                pl.ds(SLOT_OFF[10 + d - 1], 64), :
            ].astype(jnp.float32)
            rdma_x.append(start_exchange(
                13 + d, SLOT_OFF[10 + d - 1], 64, x_partner, stage=False,
                src_buf=recv_buf,
            ))
        partner_base = (1 - xbit) * 256
        for d in (0, 1, 2, 3):
            rdma_x[d].wait()
            cd = jnp.bitwise_xor(z, d)
            out_ref[pl.ds(partner_base + cd * 64, 64), :] = recv_buf[
                pl.ds(SLOT_OFF[13 + d], 64), :
            ].astype(jnp.float32)

    out2d = pl.pallas_call(
        body,
        out_shape=jax.ShapeDtypeStruct((N_ROWS, D_MODEL), jnp.float32),
        in_specs=[pl.BlockSpec(memory_space=pltpu.VMEM)] * 5,
        out_specs=pl.BlockSpec(memory_space=pltpu.VMEM),
        scratch_shapes=[
            pltpu.VMEM((SCRATCH_ROWS, D_MODEL), jnp.bfloat16),
            pltpu.VMEM((SCRATCH_ROWS, D_MODEL), jnp.bfloat16),
            pltpu.SemaphoreType.DMA((N_EXCH,)),
            pltpu.SemaphoreType.DMA((N_EXCH,)),
        ],
        compiler_params=pltpu.CompilerParams(collective_id=0),
    )(x, Wq, Wo, Kh, Vh)
    return out2d.reshape(B, SQ, D_MODEL)
